# baseline (speedup 1.0000x reference)
import jax
import jax.numpy as jnp
from jax.experimental import pallas as pl
from jax.experimental.pallas import tpu as pltpu
from jax.experimental.pallas import tpu_sc as plsc

_D = 64
_W = 80


def _gather_rows(pe, idx2d, n):
    mesh = plsc.VectorSubcoreMesh(core_axis_name="c", subcore_axis_name="s")

    @pl.kernel(
        out_type=jax.ShapeDtypeStruct((n, _D), pe.dtype),
        mesh=mesh,
        compiler_params=pltpu.CompilerParams(use_tc_tiling_on_sc=False),
    )
    def gather_kernel(pe_hbm, idx_hbm, out_hbm):
        def body(idx_vmem, out_vmem):
            pltpu.sync_copy(pe_hbm.at[idx_vmem.at[0]], out_vmem)

        nw = 32
        steps = n // (_W * nw)
        pltpu.emit_pipeline(
            body,
            grid=(nw, steps),
            in_specs=[
                pl.BlockSpec((1, _W), index_map=lambda w, j: (0, j * nw + w))
            ],
            out_specs=[
                pl.BlockSpec((_W, _D), index_map=lambda w, j: (j * nw + w, 0))
            ],
            core_axis_name=("c", "s"),
            dimension_semantics=(pltpu.PARALLEL, pltpu.ARBITRARY),
        )(idx_hbm, out_hbm)

    return gather_kernel(pe, idx2d)


def kernel(x, pe):
    b, t = x.shape
    n = b * t
    idx2d = x.reshape(1, n).astype(jnp.int32)
    out = _gather_rows(pe, idx2d, n)
    return out.reshape(b, t, _D)


# speedup vs baseline: 1.4250x; 1.4250x over previous
import jax
import jax.numpy as jnp
from jax.experimental import pallas as pl
from jax.experimental.pallas import tpu as pltpu
from jax.experimental.pallas import tpu_sc as plsc

_D = 64
_W = 64


def _gather_rows(pe, idx2d, n):
    mesh = plsc.VectorSubcoreMesh(core_axis_name="c", subcore_axis_name="s")

    @pl.kernel(
        out_type=jax.ShapeDtypeStruct((n, _D), pe.dtype),
        mesh=mesh,
        compiler_params=pltpu.CompilerParams(use_tc_tiling_on_sc=False),
    )
    def gather_kernel(pe_hbm, idx_hbm, out_hbm):
        def body(idx_vmem, out_vmem):
            pltpu.sync_copy(pe_hbm.at[idx_vmem.at[0]], out_vmem)

        nw = 32
        steps = n // (_W * nw)
        pltpu.emit_pipeline(
            body,
            grid=(nw, steps),
            in_specs=[
                pl.BlockSpec((1, _W), index_map=lambda w, j: (0, j * nw + w))
            ],
            out_specs=[
                pl.BlockSpec((_W, _D), index_map=lambda w, j: (j * nw + w, 0))
            ],
            core_axis_name=("c", "s"),
            dimension_semantics=(pltpu.PARALLEL, pltpu.ARBITRARY),
        )(idx_hbm, out_hbm)

    return gather_kernel(pe, idx2d)


def kernel(x, pe):
    b, t = x.shape
    n = b * t
    idx2d = x.reshape(1, n).astype(jnp.int32)
    out = _gather_rows(pe, idx2d, n)
    return out.reshape(b, t, _D)
